# baseline (speedup 1.0000x reference)
import jax
import jax.numpy as jnp
from jax import lax
from jax.experimental import pallas as pl
from jax.experimental.pallas import tpu as pltpu
from jax.experimental.pallas import tpu_sc as plsc

N_NODES = 10000
D = 128
ALPHA = 0.1

NC = 2
NS = 16
NW = NC * NS
CHUNK = 128
LANES = 16

N_PAD = 10240
ROWS_PER_TILE = N_PAD // NS
ZCHUNK = 128

C0_CHUNKS = 121
C1_CHUNKS = 37


def _sc_spmm_body(x_hbm, idx_hbm, out_hbm,
                  idx0, idx1, row0, row1, buf0, buf1, acc,
                  sem_i0, sem_i1, sem_g0, sem_g1, sem_s0, sem_s1):
    c = lax.axis_index("c")
    s = lax.axis_index("s")
    cnt = jnp.where(c == 0, C0_CHUNKS, C1_CHUNKS)
    start = jnp.where(c == 0, s * C0_CHUNKS, NS * C0_CHUNKS + s * C1_CHUNKS)

    idxv = (idx0, idx1)
    rowv = (row0, row1)
    bufs = (buf0, buf1)
    sem_i = (sem_i0, sem_i1)
    sem_g = (sem_g0, sem_g1)
    sem_s = (sem_s0, sem_s1)

    def issue_idx(a, b):
        pltpu.async_copy(idx_hbm.at[a], idxv[b], sem_i[b])

    def wait_idx(a, b):
        pltpu.make_async_copy(idx_hbm.at[a], idxv[b], sem_i[b]).wait()

    def issue_gather(b):
        pltpu.async_copy(x_hbm.at[idxv[b].at[0]], bufs[b], sem_g[b])

    def wait_gather(b):
        pltpu.make_async_copy(x_hbm.at[idxv[b].at[0]], bufs[b], sem_g[b]).wait()

    def scale(b):
        buf = bufs[b]

        def group_body(g2, _):
            for gg in range(4):
                g = g2 * 4 + gg
                av = idxv[b][1, pl.ds(g * LANES, LANES)]
                for i in range(LANES):
                    a = lax.bitcast_convert_type(av[i], jnp.float32)
                    r = g * LANES + i
                    for j in range(D // LANES):
                        buf[r, pl.ds(j * LANES, LANES)] = (
                            buf[r, pl.ds(j * LANES, LANES)] * a)
            return 0
        lax.fori_loop(0, CHUNK // LANES // 4, group_body, 0)

    def copy_rows(b):
        for g in range(CHUNK // LANES):
            rowv[b][pl.ds(g * LANES, LANES)] = idxv[b][2, pl.ds(g * LANES, LANES)]

    def issue_scatter(b):
        pltpu.async_copy(bufs[b], acc.at[rowv[b]], sem_s[b], add=True)

    def wait_scatter(b):
        pltpu.make_async_copy(bufs[b], acc.at[rowv[b]], sem_s[b]).wait()

    def zero_body(r, _):
        for j in range(D // LANES):
            buf0[r, pl.ds(j * LANES, LANES)] = jnp.zeros((LANES,), jnp.float32)
        return 0
    lax.fori_loop(0, ZCHUNK, zero_body, 0)
    for k in range(ROWS_PER_TILE // ZCHUNK):
        pltpu.sync_copy(buf0, acc.at[pl.ds(s * ROWS_PER_TILE + k * ZCHUNK, ZCHUNK)])
    plsc.subcore_barrier()

    pltpu.sync_copy(idx_hbm.at[start], idx0)
    issue_gather(0)
    issue_idx(start + 1, 1)

    def process(a, b, first, nxt, nxt2):
        wait_gather(b)
        if not first:
            wait_scatter(1 - b)
        if nxt:
            wait_idx(a + 1, 1 - b)
            issue_gather(1 - b)
        scale(b)
        copy_rows(b)
        issue_scatter(b)
        if nxt2:
            issue_idx(a + 2, b)

    def pair_body(p, _):
        process(start + 2 * p + 2, 0, False, True, True)
        process(start + 2 * p + 3, 1, False, True, True)
        return 0
    process(start, 0, True, True, True)
    process(start + 1, 1, False, True, True)
    lax.fori_loop(0, (cnt - 5) // 2, pair_body, 0)
    process(start + cnt - 3, 0, False, True, True)
    process(start + cnt - 2, 1, False, True, False)
    process(start + cnt - 1, 0, False, False, False)
    wait_scatter(0)

    plsc.subcore_barrier()
    for k in range(ROWS_PER_TILE // ZCHUNK):
        r0 = s * ROWS_PER_TILE + k * ZCHUNK
        pltpu.sync_copy(acc.at[pl.ds(r0, ZCHUNK)],
                        out_hbm.at[c, pl.ds(r0, ZCHUNK)])


@jax.jit
def _sc_spmm(x, idx):
    mesh = plsc.VectorSubcoreMesh(core_axis_name="c", subcore_axis_name="s")
    f = pl.kernel(
        _sc_spmm_body,
        out_type=jax.ShapeDtypeStruct((NC, N_PAD, D), jnp.float32),
        mesh=mesh,
        scratch_types=[
            pltpu.VMEM((3, CHUNK), jnp.int32),
            pltpu.VMEM((3, CHUNK), jnp.int32),
            pltpu.VMEM((CHUNK,), jnp.int32),
            pltpu.VMEM((CHUNK,), jnp.int32),
            pltpu.VMEM((CHUNK, D), jnp.float32),
            pltpu.VMEM((CHUNK, D), jnp.float32),
            pltpu.VMEM_SHARED((N_PAD, D), jnp.float32),
            pltpu.SemaphoreType.DMA,
            pltpu.SemaphoreType.DMA,
            pltpu.SemaphoreType.DMA,
            pltpu.SemaphoreType.DMA,
            pltpu.SemaphoreType.DMA,
            pltpu.SemaphoreType.DMA,
        ],
    )
    return f(x, idx)


def _tc_body(p_ref, ix_ref, w_ref, o_ref):
    hidden = (1.0 - ALPHA) * (p_ref[0] + p_ref[1]) + ALPHA * ix_ref[...]
    o_ref[...] = jnp.dot(hidden, w_ref[...], preferred_element_type=jnp.float32)


@jax.jit
def _tc_combine_matmul(partials, init_x, weight):
    blk = 1000
    grid = (N_NODES // blk,)
    return pl.pallas_call(
        _tc_body,
        grid=grid,
        in_specs=[
            pl.BlockSpec((NC, blk, D), lambda i: (0, i, 0)),
            pl.BlockSpec((blk, D), lambda i: (i, 0)),
            pl.BlockSpec((D, D), lambda i: (0, 0)),
        ],
        out_specs=pl.BlockSpec((blk, D), lambda i: (i, 0)),
        out_shape=jax.ShapeDtypeStruct((N_NODES, D), jnp.float32),
    )(partials, init_x, weight)


def kernel(x, edge_index, edge_attr, init_x, weight):
    e = edge_index.shape[1]
    t = NS * (C0_CHUNKS + C1_CHUNKS)
    ep = t * CHUNK
    assert ep >= e, "edge chunk budget too small for edge count"
    pad = ep - e
    row = jnp.pad(jnp.asarray(edge_index[0], jnp.int32), (0, pad))
    col = jnp.pad(jnp.asarray(edge_index[1], jnp.int32), (0, pad))
    ea = jnp.pad(jnp.asarray(edge_attr, jnp.float32), (0, pad))
    ea_bits = lax.bitcast_convert_type(ea, jnp.int32)
    idx = jnp.stack(
        [col.reshape(t, CHUNK),
         ea_bits.reshape(t, CHUNK),
         row.reshape(t, CHUNK)], axis=1)
    partials = _sc_spmm(x, idx)
    return _tc_combine_matmul(partials, init_x, weight)


# speedup vs baseline: 1.0107x; 1.0107x over previous
import jax
import jax.numpy as jnp
from jax import lax
from jax.experimental import pallas as pl
from jax.experimental.pallas import tpu as pltpu
from jax.experimental.pallas import tpu_sc as plsc

N_NODES = 10000
D = 128
ALPHA = 0.1

NC = 2
NS = 16
NW = NC * NS
CHUNK = 128
LANES = 16

N_PAD = 10240
ROWS_PER_TILE = N_PAD // NS
ZCHUNK = 128

C0_CHUNKS = 121
C1_CHUNKS = 37


def _sc_spmm_body(x_hbm, idx_hbm, out_hbm,
                  idx0, idx1, row0, row1, buf0, buf1, acc,
                  sem_i0, sem_i1, sem_g0, sem_g1, sem_s0, sem_s1):
    c = lax.axis_index("c")
    s = lax.axis_index("s")
    cnt = jnp.where(c == 0, C0_CHUNKS, C1_CHUNKS)
    start = jnp.where(c == 0, s * C0_CHUNKS, NS * C0_CHUNKS + s * C1_CHUNKS)

    idxv = (idx0, idx1)
    rowv = (row0, row1)
    bufs = (buf0, buf1)
    sem_i = (sem_i0, sem_i1)
    sem_g = (sem_g0, sem_g1)
    sem_s = (sem_s0, sem_s1)

    def issue_idx(a, b):
        pltpu.async_copy(idx_hbm.at[a], idxv[b], sem_i[b])

    def wait_idx(a, b):
        pltpu.make_async_copy(idx_hbm.at[a], idxv[b], sem_i[b]).wait()

    def issue_gather(b):
        pltpu.async_copy(x_hbm.at[idxv[b].at[0]], bufs[b], sem_g[b])

    def wait_gather(b):
        pltpu.make_async_copy(x_hbm.at[idxv[b].at[0]], bufs[b], sem_g[b]).wait()

    def scale(b):
        buf = bufs[b]

        def group_body(g, _):
            av = idxv[b][1, pl.ds(g * LANES, LANES)]
            for i in range(LANES):
                a = lax.bitcast_convert_type(av[i], jnp.float32)
                r = g * LANES + i
                for j in range(D // LANES):
                    buf[r, pl.ds(j * LANES, LANES)] = (
                        buf[r, pl.ds(j * LANES, LANES)] * a)
            return 0
        lax.fori_loop(0, CHUNK // LANES, group_body, 0)

    def copy_rows(b):
        for g in range(CHUNK // LANES):
            rowv[b][pl.ds(g * LANES, LANES)] = idxv[b][2, pl.ds(g * LANES, LANES)]

    def issue_scatter(b):
        pltpu.async_copy(bufs[b], acc.at[rowv[b]], sem_s[b], add=True)

    def wait_scatter(b):
        pltpu.make_async_copy(bufs[b], acc.at[rowv[b]], sem_s[b]).wait()

    def zero_body(r, _):
        for j in range(D // LANES):
            buf0[r, pl.ds(j * LANES, LANES)] = jnp.zeros((LANES,), jnp.float32)
        return 0
    lax.fori_loop(0, ZCHUNK, zero_body, 0)
    for k in range(ROWS_PER_TILE // ZCHUNK):
        pltpu.sync_copy(buf0, acc.at[pl.ds(s * ROWS_PER_TILE + k * ZCHUNK, ZCHUNK)])
    plsc.subcore_barrier()

    pltpu.sync_copy(idx_hbm.at[start], idx0)
    issue_gather(0)
    issue_idx(start + 1, 1)

    def process(a, b, first, nxt, nxt2):
        wait_gather(b)
        if not first:
            wait_scatter(1 - b)
        if nxt:
            wait_idx(a + 1, 1 - b)
            issue_gather(1 - b)
        scale(b)
        copy_rows(b)
        issue_scatter(b)
        if nxt2:
            issue_idx(a + 2, b)

    def pair_body(p, _):
        process(start + 2 * p + 2, 0, False, True, True)
        process(start + 2 * p + 3, 1, False, True, True)
        return 0
    process(start, 0, True, True, True)
    process(start + 1, 1, False, True, True)
    lax.fori_loop(0, (cnt - 5) // 2, pair_body, 0)
    process(start + cnt - 3, 0, False, True, True)
    process(start + cnt - 2, 1, False, True, False)
    process(start + cnt - 1, 0, False, False, False)
    wait_scatter(0)

    plsc.subcore_barrier()
    for k in range(ROWS_PER_TILE // ZCHUNK):
        r0 = s * ROWS_PER_TILE + k * ZCHUNK
        pltpu.sync_copy(acc.at[pl.ds(r0, ZCHUNK)],
                        out_hbm.at[c, pl.ds(r0, ZCHUNK)])


@jax.jit
def _sc_spmm(x, idx):
    mesh = plsc.VectorSubcoreMesh(core_axis_name="c", subcore_axis_name="s")
    f = pl.kernel(
        _sc_spmm_body,
        out_type=jax.ShapeDtypeStruct((NC, N_PAD, D), jnp.float32),
        mesh=mesh,
        scratch_types=[
            pltpu.VMEM((3, CHUNK), jnp.int32),
            pltpu.VMEM((3, CHUNK), jnp.int32),
            pltpu.VMEM((CHUNK,), jnp.int32),
            pltpu.VMEM((CHUNK,), jnp.int32),
            pltpu.VMEM((CHUNK, D), jnp.float32),
            pltpu.VMEM((CHUNK, D), jnp.float32),
            pltpu.VMEM_SHARED((N_PAD, D), jnp.float32),
            pltpu.SemaphoreType.DMA,
            pltpu.SemaphoreType.DMA,
            pltpu.SemaphoreType.DMA,
            pltpu.SemaphoreType.DMA,
            pltpu.SemaphoreType.DMA,
            pltpu.SemaphoreType.DMA,
        ],
    )
    return f(x, idx)


def _tc_body(p_ref, ix_ref, w_ref, o_ref):
    hidden = (1.0 - ALPHA) * (p_ref[0] + p_ref[1]) + ALPHA * ix_ref[...]
    o_ref[...] = jnp.dot(hidden, w_ref[...], preferred_element_type=jnp.float32)


@jax.jit
def _tc_combine_matmul(partials, init_x, weight):
    blk = 1000
    grid = (N_NODES // blk,)
    return pl.pallas_call(
        _tc_body,
        grid=grid,
        in_specs=[
            pl.BlockSpec((NC, blk, D), lambda i: (0, i, 0)),
            pl.BlockSpec((blk, D), lambda i: (i, 0)),
            pl.BlockSpec((D, D), lambda i: (0, 0)),
        ],
        out_specs=pl.BlockSpec((blk, D), lambda i: (i, 0)),
        out_shape=jax.ShapeDtypeStruct((N_NODES, D), jnp.float32),
    )(partials, init_x, weight)


def kernel(x, edge_index, edge_attr, init_x, weight):
    e = edge_index.shape[1]
    t = NS * (C0_CHUNKS + C1_CHUNKS)
    ep = t * CHUNK
    assert ep >= e, "edge chunk budget too small for edge count"
    pad = ep - e
    row = jnp.pad(jnp.asarray(edge_index[0], jnp.int32), (0, pad))
    col = jnp.pad(jnp.asarray(edge_index[1], jnp.int32), (0, pad))
    ea = jnp.pad(jnp.asarray(edge_attr, jnp.float32), (0, pad))
    ea_bits = lax.bitcast_convert_type(ea, jnp.int32)
    idx = jnp.stack(
        [col.reshape(t, CHUNK),
         ea_bits.reshape(t, CHUNK),
         row.reshape(t, CHUNK)], axis=1)
    partials = _sc_spmm(x, idx)
    return _tc_combine_matmul(partials, init_x, weight)
